# baseline (speedup 1.0000x reference)
import jax
import jax.numpy as jnp
from jax import lax
from jax.experimental import pallas as pl
from jax.experimental.pallas import tpu as pltpu
from jax.experimental.pallas import tpu_sc as plsc

VOCAB = 76
SEQ = 2048
DMODEL = 512
BATCH = 64

NCORES = 2
NSUB = 16
NW = NCORES * NSUB
TPS = SEQ // NW
HB = TPS // 2
NLANE = 16


def _pe_body(o_ref):
    r = lax.broadcasted_iota(jnp.int32, (SEQ, DMODEL), 0).astype(jnp.float32)
    c = lax.broadcasted_iota(jnp.int32, (SEQ, DMODEL), 1)
    even = (c - lax.rem(c, 2)).astype(jnp.float32)
    denom = jnp.exp(even * (jnp.log(10000.0) / DMODEL))
    theta = r / denom
    o_ref[...] = jnp.where(lax.rem(c, 2) == 0, jnp.sin(theta), jnp.cos(theta))


_pe_table = pl.pallas_call(
    _pe_body,
    out_shape=jax.ShapeDtypeStruct((SEQ, DMODEL), jnp.float32),
)


def _sc_body(idx_hbm, table_hbm, pe_hbm, out_hbm,
             table_v, pe_v, idx_v, ob0, ob1, sem0, sem1):
    cid = lax.axis_index("c")
    sid = lax.axis_index("s")
    wid = sid * NCORES + cid
    pltpu.sync_copy(table_hbm, table_v)
    pltpu.sync_copy(pe_hbm.at[wid], pe_v)
    pltpu.sync_copy(idx_hbm.at[wid], idx_v)

    bufs = ((ob0, sem0), (ob1, sem1))

    UJ = 8

    def compute(b, h, ob):
        def g_body(g, carry):
            tokvec = idx_v[b, pl.ds(h * HB + g * NLANE, NLANE)]

            def j_body(j0, carry2):
                for l in range(NLANE):
                    tok = tokvec[l]
                    si = h * HB + g * NLANE + l
                    row = g * NLANE + l
                    for ju in range(UJ):
                        sl = pl.ds((j0 * UJ + ju) * NLANE, NLANE)
                        ob[row, sl] = table_v[tok, sl] + pe_v[si, sl]
                return carry2

            lax.fori_loop(0, DMODEL // NLANE // UJ, j_body, 0)
            return carry

        lax.fori_loop(0, HB // NLANE, g_body, 0)

    def out_dst(b, h):
        return out_hbm.at[b, wid, pl.ds(h * HB, HB)]

    def b_body(b, carry):
        for h, (ob, sem) in enumerate(bufs):
            @pl.when(b > 0)
            def _wait():
                pltpu.make_async_copy(ob, out_dst(b - 1, h), sem).wait()

            compute(b, h, ob)
            pltpu.async_copy(ob, out_dst(b, h), sem)
        return carry

    lax.fori_loop(0, BATCH, b_body, 0)

    for h, (ob, sem) in enumerate(bufs):
        pltpu.make_async_copy(ob, out_dst(BATCH - 1, h), sem).wait()


_sc_embed = pl.kernel(
    _sc_body,
    out_type=jax.ShapeDtypeStruct((BATCH, NW, TPS, DMODEL), jnp.float32),
    mesh=plsc.VectorSubcoreMesh(core_axis_name="c", subcore_axis_name="s",
                                num_cores=NCORES, num_subcores=NSUB),
    scratch_types=[
        pltpu.VMEM((VOCAB, DMODEL), jnp.float32),
        pltpu.VMEM((TPS, DMODEL), jnp.float32),
        pltpu.VMEM((BATCH, TPS), jnp.int32),
        pltpu.VMEM((HB, DMODEL), jnp.float32),
        pltpu.VMEM((HB, DMODEL), jnp.float32),
        pltpu.SemaphoreType.DMA,
        pltpu.SemaphoreType.DMA,
    ],
)


def kernel(x, embedding):
    idx = x.astype(jnp.int32).reshape(BATCH, NW, TPS).transpose(1, 0, 2)
    pe = _pe_table().reshape(NW, TPS, DMODEL)
    out = _sc_embed(idx, embedding, pe)
    return out.reshape(BATCH, SEQ, DMODEL)


# speedup vs baseline: 1.5398x; 1.5398x over previous
import jax
import jax.numpy as jnp
from jax import lax
from jax.experimental import pallas as pl
from jax.experimental.pallas import tpu as pltpu
from jax.experimental.pallas import tpu_sc as plsc

VOCAB = 76
SEQ = 2048
DMODEL = 512
BATCH = 64

NCORES = 2
NSUB = 16
NW = NCORES * NSUB
TPS = SEQ // NW
HB = TPS // 2
NLANE = 16


def _pe_body(o_ref):
    r = lax.broadcasted_iota(jnp.int32, (SEQ, DMODEL), 0).astype(jnp.float32)
    c = lax.broadcasted_iota(jnp.int32, (SEQ, DMODEL), 1)
    even = (c - lax.rem(c, 2)).astype(jnp.float32)
    denom = jnp.exp(even * (jnp.log(10000.0) / DMODEL))
    theta = r / denom
    o_ref[...] = jnp.where(lax.rem(c, 2) == 0, jnp.sin(theta), jnp.cos(theta))


_pe_table = pl.pallas_call(
    _pe_body,
    out_shape=jax.ShapeDtypeStruct((SEQ, DMODEL), jnp.float32),
)


def _sc_body(idx_hbm, table_hbm, pe_hbm, out_hbm,
             pe_v, idx_v, gb0, gb1, ob0, ob1, gsem0, gsem1, osem0, osem1):
    cid = lax.axis_index("c")
    sid = lax.axis_index("s")
    wid = sid * NCORES + cid
    pltpu.sync_copy(pe_hbm.at[wid], pe_v)
    pltpu.sync_copy(idx_hbm.at[wid], idx_v)

    gbufs = ((gb0, gsem0), (gb1, gsem1))
    obufs = ((ob0, osem0), (ob1, osem1))

    def gsrc(b, h):
        return table_hbm.at[idx_v.at[b, pl.ds(h * HB, HB)]]

    def out_dst(b, h):
        return out_hbm.at[b, wid, pl.ds(h * HB, HB)]

    def compute(h, gb, ob):
        def r_body(r, carry):
            for j in range(DMODEL // NLANE):
                sl = pl.ds(j * NLANE, NLANE)
                ob[r, sl] = gb[r, sl] + pe_v[h * HB + r, sl]
            return carry

        lax.fori_loop(0, HB, r_body, 0)

    pltpu.async_copy(gsrc(0, 0), gb0, gsem0)

    def b_body(b, carry):
        for h in range(2):
            gb, gsem = gbufs[h]
            ob, osem = obufs[h]

            if h == 0:
                pltpu.async_copy(gsrc(b, 1), gb1, gsem1)
            else:
                @pl.when(b + 1 < BATCH)
                def _pref():
                    pltpu.async_copy(gsrc(b + 1, 0), gb0, gsem0)

            pltpu.make_async_copy(gsrc(b, h), gb, gsem).wait()

            @pl.when(b > 0)
            def _wait_out():
                pltpu.make_async_copy(ob, out_dst(b - 1, h), osem).wait()

            compute(h, gb, ob)
            pltpu.async_copy(ob, out_dst(b, h), osem)
        return carry

    lax.fori_loop(0, BATCH, b_body, 0)

    for h, (ob, osem) in enumerate(obufs):
        pltpu.make_async_copy(ob, out_dst(BATCH - 1, h), osem).wait()


_sc_embed = pl.kernel(
    _sc_body,
    out_type=jax.ShapeDtypeStruct((BATCH, NW, TPS, DMODEL), jnp.float32),
    mesh=plsc.VectorSubcoreMesh(core_axis_name="c", subcore_axis_name="s",
                                num_cores=NCORES, num_subcores=NSUB),
    scratch_types=[
        pltpu.VMEM((TPS, DMODEL), jnp.float32),
        pltpu.VMEM((BATCH, TPS), jnp.int32),
        pltpu.VMEM((HB, DMODEL), jnp.float32),
        pltpu.VMEM((HB, DMODEL), jnp.float32),
        pltpu.VMEM((HB, DMODEL), jnp.float32),
        pltpu.VMEM((HB, DMODEL), jnp.float32),
        pltpu.SemaphoreType.DMA,
        pltpu.SemaphoreType.DMA,
        pltpu.SemaphoreType.DMA,
        pltpu.SemaphoreType.DMA,
    ],
)


def kernel(x, embedding):
    idx = x.astype(jnp.int32).reshape(BATCH, NW, TPS).transpose(1, 0, 2)
    pe = _pe_table().reshape(NW, TPS, DMODEL)
    out = _sc_embed(idx, embedding, pe)
    return out.reshape(BATCH, SEQ, DMODEL)


# speedup vs baseline: 2.4283x; 1.5770x over previous
import jax
import jax.numpy as jnp
from jax import lax
from jax.experimental import pallas as pl
from jax.experimental.pallas import tpu as pltpu
from jax.experimental.pallas import tpu_sc as plsc

VOCAB = 76
SEQ = 2048
DMODEL = 512
BATCH = 64

NCORES = 2
NSUB = 16
NW = NCORES * NSUB
TPS = SEQ // NW
HB = TPS // 2
NLANE = 16


def _pe_body(o_ref):
    r = lax.broadcasted_iota(jnp.int32, (SEQ, DMODEL), 0).astype(jnp.float32)
    c = lax.broadcasted_iota(jnp.int32, (SEQ, DMODEL), 1)
    even = (c - lax.rem(c, 2)).astype(jnp.float32)
    denom = jnp.exp(even * (jnp.log(10000.0) / DMODEL))
    theta = r / denom
    o_ref[...] = jnp.where(lax.rem(c, 2) == 0, jnp.sin(theta), jnp.cos(theta))


_pe_table = pl.pallas_call(
    _pe_body,
    out_shape=jax.ShapeDtypeStruct((SEQ, DMODEL), jnp.float32),
)


def _sc_body(idx_hbm, table_hbm, pe_hbm, out_hbm,
             pe_v, idx_v, gb0, gb1, ob0, ob1, gsem0, gsem1, osem0, osem1):
    cid = lax.axis_index("c")
    sid = lax.axis_index("s")
    wid = sid * NCORES + cid
    pltpu.sync_copy(pe_hbm.at[wid], pe_v)
    pltpu.sync_copy(idx_hbm.at[wid], idx_v)

    gbufs = ((gb0, gsem0), (gb1, gsem1))
    obufs = ((ob0, osem0), (ob1, osem1))

    def gsrc(b, h):
        return table_hbm.at[idx_v.at[b, pl.ds(h * HB, HB)]]

    def out_dst(b, h):
        return out_hbm.at[b, wid, pl.ds(h * HB, HB)]

    def compute(h, gb, ob):
        def r_body(r, carry):
            for j in range(DMODEL // NLANE):
                sl = pl.ds(j * NLANE, NLANE)
                ob[r, sl] = gb[r, sl] + pe_v[h * HB + r, sl]
            return carry

        lax.fori_loop(0, HB, r_body, 0)

    pltpu.async_copy(gsrc(0, 0), gb0, gsem0)

    def b_body(b, carry):
        for h in range(2):
            gb, gsem = gbufs[h]
            ob, osem = obufs[h]

            if h == 0:
                pltpu.async_copy(gsrc(b, 1), gb1, gsem1)
            else:
                @pl.when(b + 1 < BATCH)
                def _pref():
                    pltpu.async_copy(gsrc(b + 1, 0), gb0, gsem0)

            pltpu.make_async_copy(gsrc(b, h), gb, gsem).wait()

            @pl.when(b > 0)
            def _wait_out():
                pltpu.make_async_copy(ob, out_dst(b - 1, h), osem).wait()

            compute(h, gb, ob)
            pltpu.async_copy(ob, out_dst(b, h), osem)
        return carry

    lax.fori_loop(0, BATCH, b_body, 0)

    for h, (ob, osem) in enumerate(obufs):
        pltpu.make_async_copy(ob, out_dst(BATCH - 1, h), osem).wait()


_sc_embed = pl.kernel(
    _sc_body,
    out_type=jax.ShapeDtypeStruct((BATCH, NW, TPS, DMODEL), jnp.float32),
    mesh=plsc.VectorSubcoreMesh(core_axis_name="c", subcore_axis_name="s",
                                num_cores=NCORES, num_subcores=NSUB),
    scratch_types=[
        pltpu.VMEM((TPS, DMODEL), jnp.float32),
        pltpu.VMEM((BATCH, TPS), jnp.int32),
        pltpu.VMEM((HB, DMODEL), jnp.float32),
        pltpu.VMEM((HB, DMODEL), jnp.float32),
        pltpu.VMEM((HB, DMODEL), jnp.float32),
        pltpu.VMEM((HB, DMODEL), jnp.float32),
        pltpu.SemaphoreType.DMA,
        pltpu.SemaphoreType.DMA,
        pltpu.SemaphoreType.DMA,
        pltpu.SemaphoreType.DMA,
    ],
)


def kernel(x, embedding):
    idx = x.astype(jnp.int32).reshape(BATCH, NW, TPS).transpose(1, 0, 2)
    idx = idx + (jnp.arange(NW, dtype=jnp.int32) * VOCAB)[:, None, None]
    table_rep = jnp.broadcast_to(embedding[None], (NW, VOCAB, DMODEL))
    table_rep = table_rep.reshape(NW * VOCAB, DMODEL)
    pe = _pe_table().reshape(NW, TPS, DMODEL)
    out = _sc_embed(idx, table_rep, pe)
    return out.reshape(BATCH, SEQ, DMODEL)
